# Initial kernel scaffold; baseline (speedup 1.0000x reference)
import jax
import jax.numpy as jnp
from jax.experimental import pallas as pl


def kernel(x_o, x_a, W_o1, b_o1, W_s1, b_s1, W_o2, b_o2, W_s2, b_s2, disc_W, disc_b, dec1_W, dec1_b, dec2_W, dec2_b, edge_index, edge_index2, idx):
    raise NotImplementedError("write your pallas kernel here")


# speedup vs baseline: 12.7628x; 12.7628x over previous
import functools

import jax
import jax.numpy as jnp
from jax import lax
from jax.experimental import pallas as pl
from jax.experimental.pallas import tpu as pltpu
from jax.experimental.pallas import tpu_sc as plsc

N = 10000
E = 320000
FEAT = 128
H1 = 128
H2 = 64
DEC1 = 128
P = 4096

NC = 2
NS = 16
NW = NC * NS

EPT = E // NS
CH = 80
NCHUNK = EPT // CH

RCH = 80
FULL_CHUNKS = 5
EXTRA_CHUNKS = 3

_mesh = plsc.VectorSubcoreMesh(core_axis_name="c", subcore_axis_name="s",
                               num_cores=NC, num_subcores=NS)


def _acc_chunks(s, fn):
    base = s * (RCH * (FULL_CHUNKS + EXTRA_CHUNKS))
    for k in range(FULL_CHUNKS + EXTRA_CHUNKS):
        if k < FULL_CHUNKS:
            fn(base + k * RCH)
        else:
            @pl.when(s < NS - 1)
            def _():
                fn(base + k * RCH)


def _zero_vmem_2d(buf, rows, width):
    def zrow(i, car):
        for l in range(width // 16):
            buf[i, pl.ds(l * 16, 16)] = jnp.zeros((16,), jnp.float32)
        return car
    lax.fori_loop(0, rows, zrow, 0)


def _deg_body(dst1, dst2, deg1, deg2, dstv, onesv, zbuf, wbuf, acc):
    c = lax.axis_index("c")
    s = lax.axis_index("s")
    for k in range(CH // 16):
        onesv[pl.ds(k * 16, 16)] = jnp.ones((16,), jnp.float32)
        zbuf[pl.ds(k * 16, 16)] = jnp.zeros((16,), jnp.float32)

    def run(dstr, out):
        _acc_chunks(s, lambda off: pltpu.sync_copy(
            zbuf, acc.at[pl.ds(off, RCH)]))
        plsc.subcore_barrier()
        pltpu.sync_copy(dstr.at[s], dstv)

        def chunk(j, car):
            pltpu.sync_copy(onesv, acc.at[dstv.at[j]], add=True)
            return car
        lax.fori_loop(0, NCHUNK, chunk, 0)
        plsc.subcore_barrier()

        def wb(off):
            pltpu.sync_copy(acc.at[pl.ds(off, RCH)], wbuf)
            pltpu.sync_copy(wbuf, out.at[pl.ds(off, RCH)])
        _acc_chunks(s, wb)
        plsc.subcore_barrier()

    @pl.when(c == 0)
    def _():
        run(dst1, deg1)

    @pl.when(c == 1)
    def _():
        run(dst2, deg2)


_sc_deg = functools.partial(
    pl.kernel,
    out_type=(jax.ShapeDtypeStruct((N,), jnp.float32),
              jax.ShapeDtypeStruct((N,), jnp.float32)),
    mesh=_mesh,
    scratch_types=[
        pltpu.VMEM((NCHUNK, CH), jnp.int32),
        pltpu.VMEM((CH,), jnp.float32),
        pltpu.VMEM((RCH,), jnp.float32),
        pltpu.VMEM((RCH,), jnp.float32),
        pltpu.VMEM_SHARED((N,), jnp.float32),
    ],
)(_deg_body)


def _half_pass(tbl, out, s, srcv, dstv, rows, zbuf, acc):
    _acc_chunks(s, lambda off: pltpu.sync_copy(
        zbuf, acc.at[pl.ds(off, RCH)]))
    plsc.subcore_barrier()

    def chunk(j, car):
        pltpu.sync_copy(tbl.at[srcv.at[j]], rows)
        pltpu.sync_copy(rows, acc.at[dstv.at[j]], add=True)
        return car
    lax.fori_loop(0, NCHUNK, chunk, 0)
    plsc.subcore_barrier()

    def wb(off):
        pltpu.sync_copy(acc.at[pl.ds(off, RCH)], rows)
        pltpu.sync_copy(rows, out.at[pl.ds(off, RCH)])
    _acc_chunks(s, wb)
    plsc.subcore_barrier()


HH = H1 // 2


def _l1_body(src1, dst1, src2, dst2,
             t1l, t1al, t2l, t2al, t1r, t1ar, t2r, t2ar,
             o1l, o1al, o2l, o2al, o1r, o1ar, o2r, o2ar,
             srcv, dstv, rows, zbuf, acc):
    c = lax.axis_index("c")
    s = lax.axis_index("s")
    _zero_vmem_2d(zbuf, RCH, HH)

    def run4(tA, tB, tC, tD, oA, oB, oC, oD):
        pltpu.sync_copy(src1.at[s], srcv)
        pltpu.sync_copy(dst1.at[s], dstv)
        _half_pass(tA, oA, s, srcv, dstv, rows, zbuf, acc)
        _half_pass(tB, oB, s, srcv, dstv, rows, zbuf, acc)
        pltpu.sync_copy(src2.at[s], srcv)
        pltpu.sync_copy(dst2.at[s], dstv)
        _half_pass(tC, oC, s, srcv, dstv, rows, zbuf, acc)
        _half_pass(tD, oD, s, srcv, dstv, rows, zbuf, acc)

    @pl.when(c == 0)
    def _():
        run4(t1l, t1al, t2l, t2al, o1l, o1al, o2l, o2al)

    @pl.when(c == 1)
    def _():
        run4(t1r, t1ar, t2r, t2ar, o1r, o1ar, o2r, o2ar)


_sc_fields_l1 = pl.kernel(
    _l1_body,
    out_type=tuple(jax.ShapeDtypeStruct((N, HH), jnp.float32)
                   for _ in range(8)),
    mesh=_mesh,
    compiler_params=pltpu.CompilerParams(use_tc_tiling_on_sc=False),
    scratch_types=[
        pltpu.VMEM((NCHUNK, CH), jnp.int32),
        pltpu.VMEM((NCHUNK, CH), jnp.int32),
        pltpu.VMEM((RCH, HH), jnp.float32),
        pltpu.VMEM((RCH, HH), jnp.float32),
        pltpu.VMEM_SHARED((N, HH), jnp.float32),
    ],
)


def _l2_body(src1, dst1, src2, dst2, t3o, t3a, t4o, t4a,
             o3o, o3a, o4o, o4a, srcv, dstv, rows, zbuf, acc):
    c = lax.axis_index("c")
    s = lax.axis_index("s")
    _zero_vmem_2d(zbuf, RCH, H2)

    def run2(srcr, dstr, tA, tB, oA, oB):
        pltpu.sync_copy(srcr.at[s], srcv)
        pltpu.sync_copy(dstr.at[s], dstv)
        _half_pass(tA, oA, s, srcv, dstv, rows, zbuf, acc)
        _half_pass(tB, oB, s, srcv, dstv, rows, zbuf, acc)

    @pl.when(c == 0)
    def _():
        run2(src1, dst1, t3o, t3a, o3o, o3a)

    @pl.when(c == 1)
    def _():
        run2(src2, dst2, t4o, t4a, o4o, o4a)


_sc_fields_l2 = pl.kernel(
    _l2_body,
    out_type=tuple(jax.ShapeDtypeStruct((N, H2), jnp.float32)
                   for _ in range(4)),
    mesh=_mesh,
    compiler_params=pltpu.CompilerParams(use_tc_tiling_on_sc=False),
    scratch_types=[
        pltpu.VMEM((NCHUNK, CH), jnp.int32),
        pltpu.VMEM((NCHUNK, CH), jnp.int32),
        pltpu.VMEM((RCH, H2), jnp.float32),
        pltpu.VMEM((RCH, H2), jnp.float32),
        pltpu.VMEM_SHARED((N, H2), jnp.float32),
    ],
)


def _gather_body(x2, idxr, out, idxv, rows):
    c = lax.axis_index("c")
    s = lax.axis_index("s")
    w = s * NC + c
    pltpu.sync_copy(idxr.at[w], idxv)
    for k in range(2):
        pltpu.sync_copy(x2.at[idxv.at[k]], rows)
        pltpu.sync_copy(rows, out.at[pl.ds(w * 256 + k * 128, 128)])


_sc_gather = functools.partial(
    pl.kernel,
    out_type=jax.ShapeDtypeStruct((2 * P, H1), jnp.float32),
    mesh=_mesh,
    scratch_types=[
        pltpu.VMEM((2, 128), jnp.int32),
        pltpu.VMEM((128, H1), jnp.float32),
    ],
)(_gather_body)


TB = 1000
GRID = N // TB


def _tc1_body(xo, xa, w1, w2, d1, d2,
              t1l, t1al, t2l, t2al, t1r, t1ar, t2r, t2ar):
    dinv1 = lax.rsqrt(d1[...] + 1.0)
    dinv2 = lax.rsqrt(d2[...] + 1.0)
    g1o = jnp.dot(xo[...], w1[...], preferred_element_type=jnp.float32) * dinv1
    g1a = jnp.dot(xa[...], w1[...], preferred_element_type=jnp.float32) * dinv1
    g2o = jnp.dot(xo[...], w2[...], preferred_element_type=jnp.float32) * dinv2
    g2a = jnp.dot(xa[...], w2[...], preferred_element_type=jnp.float32) * dinv2
    t1l[...], t1r[...] = g1o[:, :HH], g1o[:, HH:]
    t1al[...], t1ar[...] = g1a[:, :HH], g1a[:, HH:]
    t2l[...], t2r[...] = g2o[:, :HH], g2o[:, HH:]
    t2al[...], t2ar[...] = g2a[:, :HH], g2a[:, HH:]


def _tc2_body(o1l, o1al, o2l, o2al, o1r, o1ar, o2r, o2ar,
              t1l, t1al, t2l, t2al, t1r, t1ar, t2r, t2ar,
              d1, d2, b1, bs1, w2o, w2s, g3o, g3a, g4o, g4a):
    dinv1 = lax.rsqrt(d1[...] + 1.0)
    dinv2 = lax.rsqrt(d2[...] + 1.0)

    def enc(s1l, s1r, s2l, s2r, g1l, g1r, g2l, g2r, o3, o4):
        s1 = jnp.concatenate([s1l[...], s1r[...]], axis=1)
        g1 = jnp.concatenate([g1l[...], g1r[...]], axis=1)
        s2 = jnp.concatenate([s2l[...], s2r[...]], axis=1)
        g2 = jnp.concatenate([g2l[...], g2r[...]], axis=1)
        x1o = jax.nn.relu(dinv1 * (s1 + g1) + b1[...])
        x1s = jax.nn.relu(dinv2 * (s2 + g2) + bs1[...])
        x1 = jnp.concatenate([x1o, x1s], axis=1)
        o3[...] = jnp.dot(x1, w2o[...], preferred_element_type=jnp.float32) * dinv1
        o4[...] = jnp.dot(x1, w2s[...], preferred_element_type=jnp.float32) * dinv2

    enc(o1l, o1r, o2l, o2r, t1l, t1r, t2l, t2r, g3o, g4o)
    enc(o1al, o1ar, o2al, o2ar, t1al, t1ar, t2al, t2ar, g3a, g4a)


def _tc3_body(s3o, s3a, s4o, s4a, g3o, g3a, g4o, g4a, d1, d2, b2, bs2,
              x2, x2a, cs, csa):
    dinv1 = lax.rsqrt(d1[...] + 1.0)
    dinv2 = lax.rsqrt(d2[...] + 1.0)

    def enc(s3, s4, g3, g4, xout, csout):
        xo = dinv1 * (s3[...] + g3[...]) + b2[...]
        xs = dinv2 * (s4[...] + g4[...]) + bs2[...]
        x = jnp.concatenate([xo, xs], axis=1)
        xout[...] = x

        @pl.when(pl.program_id(0) == 0)
        def _():
            csout[...] = jnp.zeros_like(csout)
        csout[...] += jnp.sum(x, axis=0, keepdims=True)

    enc(s3o, s4o, g3o, g4o, x2, cs)
    enc(s3a, s4a, g3a, g4a, x2a, csa)


def _tc5_body(x2, x2a, cs, csa, dw, db, ro, roa):
    h = jax.nn.sigmoid(cs[...] / N)
    ha = jax.nn.sigmoid(csa[...] / N)
    dn = (((1,), (1,)), ((), ()))
    v = lax.dot_general(h, dw[...], dn, preferred_element_type=jnp.float32)
    va = lax.dot_general(ha, dw[...], dn, preferred_element_type=jnp.float32)
    r0 = lax.dot_general(x2[...], v, dn, preferred_element_type=jnp.float32)
    r1 = lax.dot_general(x2a[...], v, dn, preferred_element_type=jnp.float32)
    r2 = lax.dot_general(x2a[...], va, dn, preferred_element_type=jnp.float32)
    r3 = lax.dot_general(x2[...], va, dn, preferred_element_type=jnp.float32)
    ro[...] = jnp.concatenate([r0, r1], axis=1) + db[...]
    roa[...] = jnp.concatenate([r2, r3], axis=1) + db[...]


DB = 1024


def _tc6_body(e1, e2, w1, b1, w2, b2, out):
    a, b = e1[...], e2[...]
    f = jnp.concatenate([a + b, a * b, a, b], axis=1)
    hh = jax.nn.relu(jnp.dot(f, w1[...], preferred_element_type=jnp.float32)
                     + b1[...])
    out[...] = jnp.dot(hh, w2[...], preferred_element_type=jnp.float32) + b2[...]


def _row_spec(bw):
    return pl.BlockSpec((TB, bw), lambda i: (i, 0))


def _full_spec(shape):
    return pl.BlockSpec(shape, lambda i: tuple(0 for _ in shape))


def kernel(x_o, x_a, W_o1, b_o1, W_s1, b_s1, W_o2, b_o2, W_s2, b_s2,
           disc_W, disc_b, dec1_W, dec1_b, dec2_W, dec2_b,
           edge_index, edge_index2, idx):
    f32 = jnp.float32
    src1 = edge_index[0].reshape(NS, NCHUNK, CH)
    dst1 = edge_index[1].reshape(NS, NCHUNK, CH)
    src2 = edge_index2[0].reshape(NS, NCHUNK, CH)
    dst2 = edge_index2[1].reshape(NS, NCHUNK, CH)
    idxr = jnp.concatenate([idx[0], idx[1]]).reshape(NW, 2, 128)

    deg1, deg2 = _sc_deg(dst1, dst2)
    d1 = deg1.reshape(N, 1)
    d2 = deg2.reshape(N, 1)

    ghalves = pl.pallas_call(
        _tc1_body,
        grid=(GRID,),
        in_specs=[_row_spec(FEAT), _row_spec(FEAT),
                  _full_spec((FEAT, H1)), _full_spec((FEAT, H1)),
                  _row_spec(1), _row_spec(1)],
        out_specs=[_row_spec(HH)] * 8,
        out_shape=[jax.ShapeDtypeStruct((N, HH), f32)] * 8,
    )(x_o, x_a, W_o1, W_s1, d1, d2)

    shalves = _sc_fields_l1(src1, dst1, src2, dst2, *ghalves)

    g3o, g3a, g4o, g4a = pl.pallas_call(
        _tc2_body,
        grid=(GRID,),
        in_specs=[_row_spec(HH)] * 16 + [_row_spec(1)] * 2
                 + [_full_spec((1, H1))] * 2
                 + [_full_spec((2 * H1, H2))] * 2,
        out_specs=[_row_spec(H2)] * 4,
        out_shape=[jax.ShapeDtypeStruct((N, H2), f32)] * 4,
    )(*shalves, *ghalves, d1, d2,
      b_o1.reshape(1, H1), b_s1.reshape(1, H1), W_o2, W_s2)

    s3o, s3a, s4o, s4a = _sc_fields_l2(
        src1, dst1, src2, dst2, g3o, g3a, g4o, g4a)

    x2, x2a, cs, csa = pl.pallas_call(
        _tc3_body,
        grid=(GRID,),
        in_specs=[_row_spec(H2)] * 8 + [_row_spec(1)] * 2
                 + [_full_spec((1, H2))] * 2,
        out_specs=[_row_spec(2 * H2)] * 2
                  + [pl.BlockSpec((1, 2 * H2), lambda i: (0, 0))] * 2,
        out_shape=[jax.ShapeDtypeStruct((N, 2 * H2), f32)] * 2
                  + [jax.ShapeDtypeStruct((1, 2 * H2), f32)] * 2,
    )(s3o, s3a, s4o, s4a, g3o, g3a, g4o, g4a, d1, d2,
      b_o2.reshape(1, H2), b_s2.reshape(1, H2))

    ret_os, ret_os_a = pl.pallas_call(
        _tc5_body,
        grid=(GRID,),
        in_specs=[_row_spec(2 * H2)] * 2
                 + [_full_spec((1, 2 * H2))] * 2
                 + [_full_spec((2 * H2, 2 * H2)), _full_spec((1, 1))],
        out_specs=[_row_spec(2)] * 2,
        out_shape=[jax.ShapeDtypeStruct((N, 2), f32)] * 2,
    )(x2, x2a, cs, csa, disc_W, disc_b.reshape(1, 1))

    epairs = _sc_gather(x2, idxr)

    log = pl.pallas_call(
        _tc6_body,
        grid=(P // DB,),
        in_specs=[pl.BlockSpec((DB, H1), lambda i: (i, 0)),
                  pl.BlockSpec((DB, H1), lambda i: (i + P // DB, 0)),
                  _full_spec((4 * H1, DEC1)), _full_spec((1, DEC1)),
                  _full_spec((DEC1, 1)), _full_spec((1, 1))],
        out_specs=[pl.BlockSpec((DB, 1), lambda i: (i, 0))],
        out_shape=[jax.ShapeDtypeStruct((P, 1), f32)],
    )(epairs, epairs, dec1_W, dec1_b.reshape(1, DEC1),
      dec2_W, dec2_b.reshape(1, 1))[0]

    return (log, ret_os, ret_os_a, x2)
